# speedup vs baseline: 1.5499x; 1.5499x over previous
import jax
import jax.numpy as jnp
from jax.experimental import pallas as pl
from jax.experimental.pallas import tpu as pltpu

_INPUT_DIM = 2048
_NUM_EXPERTS = 64
_TBLK = 512


def _router_kernel(x_ref, w_ref, idx_ref, val_ref):
    logits = jax.lax.dot_general(
        x_ref[...], w_ref[...],
        dimension_numbers=(((1,), (1,)), ((), ())),
        preferred_element_type=jnp.float32,
    )

    cols = jax.lax.broadcasted_iota(jnp.int32, logits.shape, 1)
    big_i = jnp.int32(_NUM_EXPERTS)
    neg = jnp.float32(-jnp.inf)

    m1 = jnp.max(logits, axis=1, keepdims=True)
    i1 = jnp.min(jnp.where(logits == m1, cols, big_i), axis=1, keepdims=True)
    masked = jnp.where(cols == i1, neg, logits)
    m2 = jnp.max(masked, axis=1, keepdims=True)
    i2 = jnp.min(jnp.where(masked == m2, cols, big_i), axis=1, keepdims=True)

    e = jnp.exp(m2 - m1)
    s = 1.0 + e
    idx_ref[...] = jnp.concatenate([i1, i2], axis=1)
    val_ref[...] = jnp.concatenate([1.0 / s, e / s], axis=1)


def kernel(x, W):
    n_tokens = x.shape[0]
    grid = (n_tokens // _TBLK,)
    idx, val = pl.pallas_call(
        _router_kernel,
        grid=grid,
        in_specs=[
            pl.BlockSpec((_TBLK, _INPUT_DIM), lambda i: (i, 0)),
            pl.BlockSpec((_NUM_EXPERTS, _INPUT_DIM), lambda i: (0, 0)),
        ],
        out_specs=[
            pl.BlockSpec((_TBLK, 2), lambda i: (i, 0)),
            pl.BlockSpec((_TBLK, 2), lambda i: (i, 0)),
        ],
        out_shape=[
            jax.ShapeDtypeStruct((n_tokens, 2), jnp.int32),
            jax.ShapeDtypeStruct((n_tokens, 2), jnp.float32),
        ],
        compiler_params=pltpu.CompilerParams(
            dimension_semantics=("arbitrary",),
        ),
    )(x, W)
    return (idx, val)


# speedup vs baseline: 1.8202x; 1.1744x over previous
import jax
import jax.numpy as jnp
from jax.experimental import pallas as pl
from jax.experimental.pallas import tpu as pltpu

_INPUT_DIM = 2048
_NUM_EXPERTS = 64
_TBLK = 1024


def _router_kernel(x_ref, w_ref, idx_ref, val_ref):
    logits = jax.lax.dot_general(
        x_ref[...], w_ref[...],
        dimension_numbers=(((1,), (1,)), ((), ())),
        preferred_element_type=jnp.float32,
    )

    cols = jax.lax.broadcasted_iota(jnp.int32, logits.shape, 1)
    big_i = jnp.int32(_NUM_EXPERTS)
    neg = jnp.float32(-jnp.inf)

    m1 = jnp.max(logits, axis=1, keepdims=True)
    i1 = jnp.min(jnp.where(logits == m1, cols, big_i), axis=1, keepdims=True)
    masked = jnp.where(cols == i1, neg, logits)
    m2 = jnp.max(masked, axis=1, keepdims=True)
    i2 = jnp.min(jnp.where(masked == m2, cols, big_i), axis=1, keepdims=True)

    e = jnp.exp(m2 - m1)
    s = 1.0 + e
    idx_ref[...] = jnp.concatenate([i1, i2], axis=1)
    val_ref[...] = jnp.concatenate([1.0 / s, e / s], axis=1)


def kernel(x, W):
    n_tokens = x.shape[0]
    grid = (n_tokens // _TBLK,)
    idx, val = pl.pallas_call(
        _router_kernel,
        grid=grid,
        in_specs=[
            pl.BlockSpec((_TBLK, _INPUT_DIM), lambda i: (i, 0)),
            pl.BlockSpec((_NUM_EXPERTS, _INPUT_DIM), lambda i: (0, 0)),
        ],
        out_specs=[
            pl.BlockSpec((_TBLK, 2), lambda i: (i, 0)),
            pl.BlockSpec((_TBLK, 2), lambda i: (i, 0)),
        ],
        out_shape=[
            jax.ShapeDtypeStruct((n_tokens, 2), jnp.int32),
            jax.ShapeDtypeStruct((n_tokens, 2), jnp.float32),
        ],
        compiler_params=pltpu.CompilerParams(
            dimension_semantics=("arbitrary",),
        ),
    )(x, W)
    return (idx, val)


# speedup vs baseline: 1.8907x; 1.0387x over previous
import jax
import jax.numpy as jnp
from jax.experimental import pallas as pl
from jax.experimental.pallas import tpu as pltpu

_INPUT_DIM = 2048
_NUM_EXPERTS = 64
_TBLK = 2048


def _router_kernel(x_ref, w_ref, idx_ref, val_ref):
    logits = jax.lax.dot_general(
        x_ref[...], w_ref[...],
        dimension_numbers=(((1,), (1,)), ((), ())),
        preferred_element_type=jnp.float32,
    )

    cols = jax.lax.broadcasted_iota(jnp.int32, logits.shape, 1)
    big_i = jnp.int32(_NUM_EXPERTS)
    neg = jnp.float32(-jnp.inf)

    m1 = jnp.max(logits, axis=1, keepdims=True)
    i1 = jnp.min(jnp.where(logits == m1, cols, big_i), axis=1, keepdims=True)
    masked = jnp.where(cols == i1, neg, logits)
    m2 = jnp.max(masked, axis=1, keepdims=True)
    i2 = jnp.min(jnp.where(masked == m2, cols, big_i), axis=1, keepdims=True)

    e = jnp.exp(m2 - m1)
    s = 1.0 + e
    idx_ref[...] = jnp.concatenate([i1, i2], axis=1)
    val_ref[...] = jnp.concatenate([1.0 / s, e / s], axis=1)


def kernel(x, W):
    n_tokens = x.shape[0]
    grid = (n_tokens // _TBLK,)
    idx, val = pl.pallas_call(
        _router_kernel,
        grid=grid,
        in_specs=[
            pl.BlockSpec((_TBLK, _INPUT_DIM), lambda i: (i, 0)),
            pl.BlockSpec((_NUM_EXPERTS, _INPUT_DIM), lambda i: (0, 0)),
        ],
        out_specs=[
            pl.BlockSpec((_TBLK, 2), lambda i: (i, 0)),
            pl.BlockSpec((_TBLK, 2), lambda i: (i, 0)),
        ],
        out_shape=[
            jax.ShapeDtypeStruct((n_tokens, 2), jnp.int32),
            jax.ShapeDtypeStruct((n_tokens, 2), jnp.float32),
        ],
        compiler_params=pltpu.CompilerParams(
            dimension_semantics=("arbitrary",),
        ),
    )(x, W)
    return (idx, val)


# speedup vs baseline: 1.8908x; 1.0000x over previous
import jax
import jax.numpy as jnp
from jax.experimental import pallas as pl
from jax.experimental.pallas import tpu as pltpu

_INPUT_DIM = 2048
_NUM_EXPERTS = 64
_TBLK = 2048


def _router_kernel(x_ref, w_ref, idx_ref, val_ref):
    logits = jax.lax.dot_general(
        x_ref[...], w_ref[...],
        dimension_numbers=(((1,), (1,)), ((), ())),
        preferred_element_type=jnp.float32,
    )

    cols = jax.lax.broadcasted_iota(jnp.int32, logits.shape, 1)
    big_i = jnp.int32(_NUM_EXPERTS)
    neg = jnp.float32(-jnp.inf)

    m1 = jnp.max(logits, axis=1, keepdims=True)
    i1 = jnp.min(jnp.where(logits == m1, cols, big_i), axis=1, keepdims=True)
    masked = jnp.where(cols == i1, neg, logits)
    m2 = jnp.max(masked, axis=1, keepdims=True)
    i2 = jnp.min(jnp.where(masked == m2, cols, big_i), axis=1, keepdims=True)

    e = jnp.exp(m2 - m1)
    s = 1.0 + e
    idx_ref[...] = jnp.concatenate([i1, i2], axis=1)
    val_ref[...] = jnp.concatenate([1.0 / s, e / s], axis=1)


def kernel(x, W):
    n_tokens = x.shape[0]
    grid = (n_tokens // _TBLK,)
    idx, val = pl.pallas_call(
        _router_kernel,
        grid=grid,
        in_specs=[
            pl.BlockSpec((_TBLK, _INPUT_DIM), lambda i: (i, 0)),
            pl.BlockSpec((_NUM_EXPERTS, _INPUT_DIM), lambda i: (0, 0)),
        ],
        out_specs=[
            pl.BlockSpec((_TBLK, 2), lambda i: (i, 0)),
            pl.BlockSpec((_TBLK, 2), lambda i: (i, 0)),
        ],
        out_shape=[
            jax.ShapeDtypeStruct((n_tokens, 2), jnp.int32),
            jax.ShapeDtypeStruct((n_tokens, 2), jnp.float32),
        ],
        compiler_params=pltpu.CompilerParams(
            dimension_semantics=("parallel",),
        ),
    )(x, W)
    return (idx, val)
